# baseline (speedup 1.0000x reference)
import jax
import jax.numpy as jnp
from jax import lax
from jax.experimental import pallas as pl
from jax.experimental.pallas import tpu as pltpu
from jax.experimental.pallas import tpu_sc as plsc

K = 16
D = 128

_L = 16
_NC = 2
_NW = 32
_RPW = D // _NW


def _sc_body(dens_ref, conf_ref, out_ref, buf, conf_v, sem):
    wid = lax.axis_index("s") * _NC + lax.axis_index("c")
    base = wid * _RPW
    pltpu.sync_copy(conf_ref, conf_v)
    conf = conf_v[...][0]
    src = dens_ref.at[pl.ds(conf * D + base, _RPW)]
    pltpu.async_copy(src, buf, sem).wait()
    pltpu.sync_copy(buf, out_ref.at[pl.ds(base, _RPW)])


def kernel(density, conformation):
    dens3d = density.reshape(K * D, D, D)
    conf_vec = jnp.full((_L,), conformation, jnp.int32)
    mesh = plsc.VectorSubcoreMesh(core_axis_name="c", subcore_axis_name="s")
    sc_call = pl.kernel(
        _sc_body,
        out_type=jax.ShapeDtypeStruct((D, D, D), jnp.float32),
        mesh=mesh,
        scratch_types=[
            pltpu.VMEM((_RPW, D, D), jnp.float32),
            pltpu.VMEM((_L,), jnp.int32),
            pltpu.SemaphoreType.DMA,
        ],
    )
    return sc_call(dens3d, conf_vec)


# speedup vs baseline: 3.8678x; 3.8678x over previous
import jax
import jax.numpy as jnp
from jax.experimental import pallas as pl
from jax.experimental.pallas import tpu as pltpu

K = 16
D = 128


_NC = 4
_CD = D // _NC


def _select_body(conf_ref, dens_ref, out_ref, buf, rsem, wsem):
    i = conf_ref[0]
    reads = []
    writes = []
    for c in range(_NC):
        sl = pl.ds(c * _CD, _CD)
        reads.append(
            pltpu.make_async_copy(dens_ref.at[i, sl], buf.at[c], rsem.at[c])
        )
        writes.append(
            pltpu.make_async_copy(buf.at[c], out_ref.at[sl], wsem.at[c])
        )
    for r in reads:
        r.start()
    for c in range(_NC):
        reads[c].wait()
        writes[c].start()
    for w in writes:
        w.wait()


def kernel(density, conformation):
    conf = jnp.atleast_1d(jnp.asarray(conformation, jnp.int32))
    grid_spec = pltpu.PrefetchScalarGridSpec(
        num_scalar_prefetch=1,
        grid=(1,),
        in_specs=[pl.BlockSpec(memory_space=pl.ANY)],
        out_specs=pl.BlockSpec(memory_space=pl.ANY),
        scratch_shapes=[
            pltpu.VMEM((_NC, _CD, D, D), jnp.float32),
            pltpu.SemaphoreType.DMA((_NC,)),
            pltpu.SemaphoreType.DMA((_NC,)),
        ],
    )
    return pl.pallas_call(
        _select_body,
        grid_spec=grid_spec,
        out_shape=jax.ShapeDtypeStruct((D, D, D), density.dtype),
    )(conf, density)
